# Initial kernel scaffold; baseline (speedup 1.0000x reference)
import jax
import jax.numpy as jnp
from jax.experimental import pallas as pl


def kernel(x, edge, edge_weight, prev_hidden, deg, Wz, bz, Wr, br, Wh, bh, Lz, lbz, Lr, lbr, Lh, lbh, Wout, bout):
    raise NotImplementedError("write your pallas kernel here")


# speedup vs baseline: 10.6337x; 10.6337x over previous
import functools

import jax
import jax.numpy as jnp
from jax import lax
from jax.experimental import pallas as pl
from jax.experimental.pallas import tpu as pltpu
from jax.experimental.pallas import tpu_sc as plsc

_CHUNK = 80


def _sc_aggregate(x, src, dst, ew):
    n, d = x.shape
    e = src.shape[0]
    info = plsc.get_sparse_core_info()
    nc, ns, lanes = info.num_cores, info.num_subcores, info.num_lanes
    nw = nc * ns
    epw = e // nw
    nch = epw // _CHUNK
    zrows = 128
    rows_pt = 5 * zrows
    n_pad = ns * rows_pt
    mesh = plsc.VectorSubcoreMesh(core_axis_name="c", subcore_axis_name="s")

    @functools.partial(
        pl.kernel,
        mesh=mesh,
        out_type=jax.ShapeDtypeStruct((nc, n_pad, d), jnp.float32),
        scratch_types=[
            pltpu.VMEM_SHARED((n_pad, d), jnp.float32),
            pltpu.VMEM((zrows, d), jnp.float32),
            pltpu.VMEM((_CHUNK,), jnp.int32),
            pltpu.VMEM((_CHUNK,), jnp.int32),
            pltpu.VMEM((_CHUNK,), jnp.float32),
            pltpu.VMEM((_CHUNK, d), jnp.float32),
            pltpu.SemaphoreType.DMA,
        ],
    )
    def agg(x_hbm, src_hbm, dst_hbm, ew_hbm, out_hbm,
            acc, zbuf, sidx, didx, ewv, rows, sem):
        c = lax.axis_index("c")
        s = lax.axis_index("s")
        wid = s * nc + c
        row0 = s * rows_pt

        def zrow(r, carry):
            for j in range(d // lanes):
                zbuf[r, pl.ds(j * lanes, lanes)] = jnp.zeros((lanes,), jnp.float32)
            return carry
        lax.fori_loop(0, zrows, zrow, 0)
        for i in range(rows_pt // zrows):
            pltpu.sync_copy(zbuf, acc.at[pl.ds(row0 + i * zrows, zrows)])
        plsc.subcore_barrier()

        base = wid * epw

        def chunk(i, carry):
            off = base + i * _CHUNK
            pltpu.sync_copy(src_hbm.at[pl.ds(off, _CHUNK)], sidx)
            pltpu.sync_copy(dst_hbm.at[pl.ds(off, _CHUNK)], didx)
            pltpu.sync_copy(ew_hbm.at[pl.ds(off, _CHUNK)], ewv)
            pltpu.async_copy(x_hbm.at[sidx], rows, sem).wait()

            def scale(g, inner):
                w16 = ewv[pl.ds(g * lanes, lanes)]
                for l in range(lanes):
                    w = w16[l]
                    k = g * lanes + l
                    for j in range(d // lanes):
                        sl = pl.ds(j * lanes, lanes)
                        rows[k, sl] = rows[k, sl] * w
                return inner
            lax.fori_loop(0, _CHUNK // lanes, scale, 0)
            pltpu.sync_copy(rows, acc.at[didx], add=True)
            return carry
        lax.fori_loop(0, nch, chunk, 0)
        plsc.subcore_barrier()

        for i in range(rows_pt // zrows):
            r0 = row0 + i * zrows
            pltpu.sync_copy(acc.at[pl.ds(r0, zrows)], out_hbm.at[c, pl.ds(r0, zrows)])

    return agg(x, src, dst, ew)


def _tc_dense(parts, ph, degc, wz, wr, wh, bcat, lz, lr, lh, lbz, lbr, lbh,
              wout_row, bout11):
    n, d = ph.shape
    blk = 2000
    grid = n // blk

    def body(parts_ref, ph_ref, deg_ref, wz_ref, wr_ref, wh_ref, bcat_ref,
             lz_ref, lr_ref, lh_ref, lbz_ref, lbr_ref, lbh_ref,
             wout_ref, bout_ref, y_ref, h_ref):
        a = parts_ref[0] + parts_ref[1]
        an = a / deg_ref[...]
        wcat = jnp.concatenate([wz_ref[...], wr_ref[...], wh_ref[...]], axis=1)
        conv = jnp.dot(an, wcat, preferred_element_type=jnp.float32) + bcat_ref[...]
        cz, cr, ch = conv[:, :d], conv[:, d:2 * d], conv[:, 2 * d:]
        phv = ph_ref[...]
        lzm, lrm, lhm = lz_ref[...], lr_ref[...], lh_ref[...]
        z = jax.nn.sigmoid(
            jnp.dot(cz, lzm[:d], preferred_element_type=jnp.float32)
            + jnp.dot(phv, lzm[d:], preferred_element_type=jnp.float32)
            + lbz_ref[...])
        r = jax.nn.sigmoid(
            jnp.dot(cr, lrm[:d], preferred_element_type=jnp.float32)
            + jnp.dot(phv, lrm[d:], preferred_element_type=jnp.float32)
            + lbr_ref[...])
        ht = jnp.tanh(
            jnp.dot(ch, lhm[:d], preferred_element_type=jnp.float32)
            + jnp.dot(r * phv, lhm[d:], preferred_element_type=jnp.float32)
            + lbh_ref[...])
        h = z * phv + (1.0 - z) * ht
        h_ref[...] = h
        y_ref[...] = (jnp.sum(jnp.maximum(h, 0.0) * wout_ref[...],
                              axis=1, keepdims=True) + bout_ref[...])

    row_spec = pl.BlockSpec((blk, d), lambda i: (i, 0))
    full = lambda shape: pl.BlockSpec(shape, lambda i: tuple(0 for _ in shape))
    return pl.pallas_call(
        body,
        grid=(grid,),
        in_specs=[
            pl.BlockSpec((2, blk, d), lambda i: (0, i, 0)),
            row_spec,
            pl.BlockSpec((blk, 1), lambda i: (i, 0)),
            full((d, d)), full((d, d)), full((d, d)), full((1, 3 * d)),
            full((2 * d, d)), full((2 * d, d)), full((2 * d, d)),
            full((1, d)), full((1, d)), full((1, d)),
            full((1, d)), full((1, 1)),
        ],
        out_specs=[pl.BlockSpec((blk, 1), lambda i: (i, 0)), row_spec],
        out_shape=[
            jax.ShapeDtypeStruct((n, 1), jnp.float32),
            jax.ShapeDtypeStruct((n, d), jnp.float32),
        ],
    )(parts, ph, degc, wz, wr, wh, bcat, lz, lr, lh,
      lbz.reshape(1, -1), lbr.reshape(1, -1), lbh.reshape(1, -1),
      wout_row, bout11)


def kernel(x, edge, edge_weight, prev_hidden, deg,
           Wz, bz, Wr, br, Wh, bh, Lz, lbz, Lr, lbr, Lh, lbh, Wout, bout):
    parts = _sc_aggregate(x, edge[0], edge[1], edge_weight)
    bcat = jnp.concatenate([bz, br, bh]).reshape(1, -1)
    y, h = _tc_dense(parts, prev_hidden, deg.reshape(-1, 1),
                     Wz, Wr, Wh, bcat, Lz, Lr, Lh, lbz, lbr, lbh,
                     Wout.reshape(1, -1), bout.reshape(1, 1))
    return y, h


# speedup vs baseline: 22.8633x; 2.1501x over previous
import functools

import jax
import jax.numpy as jnp
from jax import lax
from jax.experimental import pallas as pl
from jax.experimental.pallas import tpu as pltpu
from jax.experimental.pallas import tpu_sc as plsc

_CHUNK = 80


def _sc_aggregate(x, src3, dst, ew, zeros):
    n, d = x.shape
    nw_in, nch, _ = src3.shape
    info = plsc.get_sparse_core_info()
    nc, ns, lanes = info.num_cores, info.num_subcores, info.num_lanes
    rows_pt = 632
    n_pad = ns * rows_pt
    mesh = plsc.VectorSubcoreMesh(core_axis_name="c", subcore_axis_name="s")

    @functools.partial(
        pl.kernel,
        mesh=mesh,
        out_type=jax.ShapeDtypeStruct((nc, n_pad, d), jnp.float32),
        scratch_types=[
            pltpu.VMEM_SHARED((n_pad, d), jnp.float32),
            pltpu.VMEM((nch, _CHUNK), jnp.int32),
            pltpu.VMEM((_CHUNK,), jnp.int32),
            pltpu.VMEM((_CHUNK,), jnp.int32),
            pltpu.VMEM((_CHUNK,), jnp.float32),
            pltpu.VMEM((_CHUNK,), jnp.float32),
            pltpu.VMEM((_CHUNK, d), jnp.float32),
            pltpu.VMEM((_CHUNK, d), jnp.float32),
            pltpu.SemaphoreType.DMA,
            pltpu.SemaphoreType.DMA,
            pltpu.SemaphoreType.DMA,
        ],
    )
    def agg(x_hbm, src_hbm, dst_hbm, ew_hbm, z_hbm, out_hbm,
            acc, sbuf, d_a, d_b, ew_a, ew_b, rows_a, rows_b, gsem, esem, ssem):
        c = lax.axis_index("c")
        s = lax.axis_index("s")
        wid = s * nc + c
        row0 = s * rows_pt
        ebase = wid * nch * _CHUNK

        pltpu.sync_copy(src_hbm.at[wid], sbuf)
        pltpu.sync_copy(z_hbm, acc.at[pl.ds(row0, rows_pt)])
        plsc.subcore_barrier()

        def gstart(ci, rows):
            return pltpu.async_copy(x_hbm.at[sbuf.at[ci]], rows, gsem)

        def estart(ci, didx, ewv):
            sl = pl.ds(ebase + ci * _CHUNK, _CHUNK)
            pltpu.async_copy(dst_hbm.at[sl], didx, esem)
            pltpu.async_copy(ew_hbm.at[sl], ewv, esem)

        def sstart(ci, rows, didx):
            return pltpu.async_copy(rows, acc.at[didx], ssem, add=True)

        def edrain(didx, ewv):
            pltpu.make_async_copy(dst_hbm.at[pl.ds(0, _CHUNK)], didx, esem).wait()
            pltpu.make_async_copy(ew_hbm.at[pl.ds(0, _CHUNK)], ewv, esem).wait()

        def drain(sem, rows):
            pltpu.make_async_copy(x_hbm.at[pl.ds(0, _CHUNK)], rows, sem).wait()

        def scale(rows, ewv):
            def grp(g, inner):
                w16 = ewv[pl.ds(g * lanes, lanes)]
                for l in range(lanes):
                    w = w16[l]
                    k = g * lanes + l
                    for j in range(d // lanes):
                        sl = pl.ds(j * lanes, lanes)
                        rows[k, sl] = rows[k, sl] * w
                return inner
            lax.fori_loop(0, _CHUNK // lanes, grp, 0)

        g0 = gstart(0, rows_a)
        estart(0, d_a, ew_a)
        g0.wait()
        edrain(d_a, ew_a)
        gstart(1, rows_b)
        estart(1, d_b, ew_b)
        scale(rows_a, ew_a)
        sstart(0, rows_a, d_a)

        def pair(p, carry):
            cb = 2 * p + 1
            drain(gsem, rows_b)
            edrain(d_b, ew_b)
            drain(ssem, rows_a)
            gstart(cb + 1, rows_a)
            estart(cb + 1, d_a, ew_a)
            scale(rows_b, ew_b)
            sstart(cb, rows_b, d_b)
            drain(gsem, rows_a)
            edrain(d_a, ew_a)
            drain(ssem, rows_b)
            @pl.when(cb + 2 < nch)
            def _():
                gstart(cb + 2, rows_b)
                estart(cb + 2, d_b, ew_b)
            scale(rows_a, ew_a)
            sstart(cb + 1, rows_a, d_a)
            return carry
        lax.fori_loop(0, (nch - 1) // 2, pair, 0)
        drain(ssem, rows_a)
        plsc.subcore_barrier()

        pltpu.sync_copy(acc.at[pl.ds(row0, rows_pt)],
                        out_hbm.at[c, pl.ds(row0, rows_pt)])

    return agg(x, src3, dst, ew, zeros)


def _tc_dense(parts, ph, degc, wz, wr, wh, bcat, lz, lr, lh, lbz, lbr, lbh,
              wout_row, bout11):
    n, d = ph.shape
    blk = 2000
    grid = n // blk

    def body(parts_ref, ph_ref, deg_ref, wz_ref, wr_ref, wh_ref, bcat_ref,
             lz_ref, lr_ref, lh_ref, lbz_ref, lbr_ref, lbh_ref,
             wout_ref, bout_ref, y_ref, h_ref):
        a = parts_ref[0] + parts_ref[1]
        an = a / deg_ref[...]
        wcat = jnp.concatenate([wz_ref[...], wr_ref[...], wh_ref[...]], axis=1)
        conv = jnp.dot(an, wcat, preferred_element_type=jnp.float32) + bcat_ref[...]
        cz, cr, ch = conv[:, :d], conv[:, d:2 * d], conv[:, 2 * d:]
        phv = ph_ref[...]
        lzm, lrm, lhm = lz_ref[...], lr_ref[...], lh_ref[...]
        z = jax.nn.sigmoid(
            jnp.dot(cz, lzm[:d], preferred_element_type=jnp.float32)
            + jnp.dot(phv, lzm[d:], preferred_element_type=jnp.float32)
            + lbz_ref[...])
        r = jax.nn.sigmoid(
            jnp.dot(cr, lrm[:d], preferred_element_type=jnp.float32)
            + jnp.dot(phv, lrm[d:], preferred_element_type=jnp.float32)
            + lbr_ref[...])
        ht = jnp.tanh(
            jnp.dot(ch, lhm[:d], preferred_element_type=jnp.float32)
            + jnp.dot(r * phv, lhm[d:], preferred_element_type=jnp.float32)
            + lbh_ref[...])
        h = z * phv + (1.0 - z) * ht
        h_ref[...] = h
        y_ref[...] = (jnp.sum(jnp.maximum(h, 0.0) * wout_ref[...],
                              axis=1, keepdims=True) + bout_ref[...])

    row_spec = pl.BlockSpec((blk, d), lambda i: (i, 0))
    full = lambda shape: pl.BlockSpec(shape, lambda i: tuple(0 for _ in shape))
    return pl.pallas_call(
        body,
        grid=(grid,),
        in_specs=[
            pl.BlockSpec((2, blk, d), lambda i: (0, i, 0)),
            row_spec,
            pl.BlockSpec((blk, 1), lambda i: (i, 0)),
            full((d, d)), full((d, d)), full((d, d)), full((1, 3 * d)),
            full((2 * d, d)), full((2 * d, d)), full((2 * d, d)),
            full((1, d)), full((1, d)), full((1, d)),
            full((1, d)), full((1, 1)),
        ],
        out_specs=[pl.BlockSpec((blk, 1), lambda i: (i, 0)), row_spec],
        out_shape=[
            jax.ShapeDtypeStruct((n, 1), jnp.float32),
            jax.ShapeDtypeStruct((n, d), jnp.float32),
        ],
    )(parts, ph, degc, wz, wr, wh, bcat, lz, lr, lh,
      lbz.reshape(1, -1), lbr.reshape(1, -1), lbh.reshape(1, -1),
      wout_row, bout11)


def kernel(x, edge, edge_weight, prev_hidden, deg,
           Wz, bz, Wr, br, Wh, bh, Lz, lbz, Lr, lbr, Lh, lbh, Wout, bout):
    e = edge.shape[1]
    nw = 32
    nch = e // (nw * _CHUNK)
    parts = _sc_aggregate(x,
                          edge[0].reshape(nw, nch, _CHUNK),
                          edge[1],
                          edge_weight,
                          jnp.zeros((632, x.shape[1]), jnp.float32))
    bcat = jnp.concatenate([bz, br, bh]).reshape(1, -1)
    y, h = _tc_dense(parts, prev_hidden, deg.reshape(-1, 1),
                     Wz, Wr, Wh, bcat, Lz, Lr, Lh, lbz, lbr, lbh,
                     Wout.reshape(1, -1), bout.reshape(1, 1))
    return y, h
